# baseline (speedup 1.0000x reference)
import functools

import jax
import jax.numpy as jnp
from jax import lax
from jax.experimental import pallas as pl
from jax.experimental.pallas import tpu as pltpu
from jax.experimental.pallas import tpu_sc as plsc

N_NODES = 10000
N_PAD = 10240
N_EDGES = 320000
HID = 64
N_GRAPHS = 128

_NC, _NS = 2, 16
_NW = _NC * _NS
_CH = 128
_CPT = 80
_E_PAD = _NW * _CPT * _CH
_NBUF = 8
_LA = 4
_RPT = N_PAD // _NS


def _edge_agg(y, src2d, dst2d):
    mesh = plsc.VectorSubcoreMesh(core_axis_name="c", subcore_axis_name="s")

    @functools.partial(
        pl.kernel,
        mesh=mesh,
        compiler_params=pltpu.CompilerParams(use_tc_tiling_on_sc=False),
        out_type=jax.ShapeDtypeStruct((_NC, N_PAD, HID), jnp.float32),
        scratch_types=[
            pltpu.VMEM((_CPT, _CH), jnp.int32),
            pltpu.VMEM((_CPT, _CH), jnp.int32),
            [pltpu.VMEM((_CH, HID), jnp.float32) for _ in range(_NBUF)],
            pltpu.VMEM_SHARED((N_PAD, HID), jnp.float32),
            [pltpu.SemaphoreType.DMA for _ in range(_NBUF)],
            [pltpu.SemaphoreType.DMA for _ in range(_NBUF)],
        ],
    )
    def agg(y_hbm, src_hbm, dst_hbm, out_hbm,
            sidx, didx, rows, acc, sem_g, sem_s):
        c = lax.axis_index("c")
        s = lax.axis_index("s")
        tid = c * _NS + s

        zeros16 = jnp.zeros((16,), jnp.float32)
        zbuf = rows[0]

        def zrow(r, _):
            for j in range(HID // 16):
                zbuf[r, pl.ds(j * 16, 16)] = zeros16
            return ()

        lax.fori_loop(0, _CH, zrow, ())
        for i in range(_RPT // _CH):
            pltpu.sync_copy(zbuf, acc.at[pl.ds(s * _RPT + i * _CH, _CH)])

        r0 = tid * _CPT
        pltpu.sync_copy(src_hbm.at[pl.ds(r0, _CPT)], sidx)
        pltpu.sync_copy(dst_hbm.at[pl.ds(r0, _CPT)], didx)
        plsc.subcore_barrier()

        def g_start(j, b):
            pltpu.async_copy(y_hbm.at[sidx.at[j]], rows[b], sem_g[b])

        def g_wait(b):
            pltpu.make_async_copy(y_hbm.at[pl.ds(0, _CH)], rows[b],
                                  sem_g[b]).wait()

        def s_start(k, b):
            pltpu.async_copy(rows[b], acc.at[didx.at[k]], sem_s[b], add=True)

        def s_wait(b):
            pltpu.make_async_copy(rows[b], acc.at[pl.ds(0, _CH)],
                                  sem_s[b]).wait()

        for j in range(_LA):
            g_start(j, j)
        for k in range(_LA):
            g_start(k + _LA, k + _LA)
            g_wait(k)
            s_start(k, k)

        def group(g, _):
            for b8 in range(_NBUF):
                k = _LA + g * _NBUF + b8
                bj = b8
                s_wait(bj)
                g_start(k + _LA, bj)
                b = (_LA + b8) % _NBUF
                g_wait(b)
                s_start(k, b)
            return ()

        lax.fori_loop(0, (_CPT - 2 * _LA) // _NBUF, group, ())

        for k in range(_CPT - _LA, _CPT):
            b = k % _NBUF
            g_wait(b)
            s_start(k, b)
        for b in range(_NBUF):
            s_wait(b)

        plsc.subcore_barrier()
        pltpu.sync_copy(acc.at[pl.ds(s * _RPT, _RPT)],
                        out_hbm.at[c].at[pl.ds(s * _RPT, _RPT)])

    return agg(y, src2d, dst2d)


_BM = 1000


def _proj_body(x_ref, w_ref, o_ref):
    o_ref[...] = jnp.dot(x_ref[...], w_ref[...],
                         preferred_element_type=jnp.float32)


def _proj(x, w):
    m, k = x.shape
    n = w.shape[1]
    return pl.pallas_call(
        _proj_body,
        grid=(N_NODES // _BM,),
        in_specs=[
            pl.BlockSpec((_BM, k), lambda i: (i, 0)),
            pl.BlockSpec((k, n), lambda i: (0, 0)),
        ],
        out_specs=pl.BlockSpec((_BM, n), lambda i: (i, 0)),
        out_shape=jax.ShapeDtypeStruct((N_PAD, n), jnp.float32),
    )(x, w)


def _mid_body(y1_ref, p_ref, b1a_ref, w1b_ref, b1b_ref, w2a_ref, o_ref):
    u = jnp.maximum(y1_ref[...] + p_ref[0] + p_ref[1] + b1a_ref[...], 0.0)
    h = jnp.maximum(
        jnp.dot(u, w1b_ref[...], preferred_element_type=jnp.float32)
        + b1b_ref[...], 0.0)
    o_ref[...] = jnp.dot(h, w2a_ref[...], preferred_element_type=jnp.float32)


def _mid(y1, p, b1a, w1b, b1b, w2a):
    return pl.pallas_call(
        _mid_body,
        grid=(N_NODES // _BM,),
        in_specs=[
            pl.BlockSpec((_BM, HID), lambda i: (i, 0)),
            pl.BlockSpec((_NC, _BM, HID), lambda i: (0, i, 0)),
            pl.BlockSpec((1, HID), lambda i: (0, 0)),
            pl.BlockSpec((HID, HID), lambda i: (0, 0)),
            pl.BlockSpec((1, HID), lambda i: (0, 0)),
            pl.BlockSpec((HID, HID), lambda i: (0, 0)),
        ],
        out_specs=pl.BlockSpec((_BM, HID), lambda i: (i, 0)),
        out_shape=jax.ShapeDtypeStruct((N_PAD, HID), jnp.float32),
    )(y1, p, b1a, w1b, b1b, w2a)


def _tail_body(y2_ref, p_ref, b2a_ref, w2b_ref, b2b_ref, batch_ref,
               wg_ref, bg_ref, wo_ref, bo_ref, g_ref, o_ref):
    i = pl.program_id(0)
    ng = pl.num_programs(0)
    v = jnp.maximum(y2_ref[...] + p_ref[0] + p_ref[1] + b2a_ref[...], 0.0)
    h2 = jnp.maximum(
        jnp.dot(v, w2b_ref[...], preferred_element_type=jnp.float32)
        + b2b_ref[...], 0.0)
    ids = batch_ref[pl.ds(i, 1), :]
    onehot_t = (jnp.broadcast_to(ids, (N_GRAPHS, _BM))
                == lax.broadcasted_iota(jnp.int32, (N_GRAPHS, _BM), 0)
                ).astype(jnp.float32)
    gpart = lax.dot_general(onehot_t, h2, (((1,), (0,)), ((), ())),
                            preferred_element_type=jnp.float32)

    @pl.when(i == 0)
    def _init():
        g_ref[...] = gpart

    @pl.when(i > 0)
    def _accum():
        g_ref[...] += gpart

    @pl.when(i == ng - 1)
    def _head():
        g = g_ref[...]
        t = jnp.maximum(
            jnp.dot(g, wg_ref[...], preferred_element_type=jnp.float32)
            + bg_ref[...], 0.0)
        o_ref[...] = (jnp.dot(t, wo_ref[...],
                              preferred_element_type=jnp.float32)
                      + bo_ref[...])


def _tail_stage(y2, p, b2a, w2b, b2b, batch2d, wg, bg, wo, bo):
    out_dim = wo.shape[1]
    nb = N_NODES // _BM
    _, out = pl.pallas_call(
        _tail_body,
        grid=(nb,),
        in_specs=[
            pl.BlockSpec((_BM, HID), lambda i: (i, 0)),
            pl.BlockSpec((_NC, _BM, HID), lambda i: (0, i, 0)),
            pl.BlockSpec((1, HID), lambda i: (0, 0)),
            pl.BlockSpec((HID, HID), lambda i: (0, 0)),
            pl.BlockSpec((1, HID), lambda i: (0, 0)),
            pl.BlockSpec((nb, _BM), lambda i: (0, 0)),
            pl.BlockSpec((HID, HID), lambda i: (0, 0)),
            pl.BlockSpec((1, HID), lambda i: (0, 0)),
            pl.BlockSpec((HID, out_dim), lambda i: (0, 0)),
            pl.BlockSpec((1, out_dim), lambda i: (0, 0)),
        ],
        out_specs=[
            pl.BlockSpec((N_GRAPHS, HID), lambda i: (0, 0)),
            pl.BlockSpec((N_GRAPHS, out_dim), lambda i: (0, 0)),
        ],
        out_shape=[
            jax.ShapeDtypeStruct((N_GRAPHS, HID), jnp.float32),
            jax.ShapeDtypeStruct((N_GRAPHS, out_dim), jnp.float32),
        ],
    )(y2, p, b2a, w2b, b2b, batch2d, wg, bg, wo, bo)
    return out


def kernel(x, edge_index, batch, W1a, b1a, W1b, b1b, W2a, b2a, W2b, b2b,
           Wg, bg, Wo, bo):
    pad = _E_PAD - N_EDGES
    src = jnp.concatenate(
        [edge_index[0].astype(jnp.int32), jnp.zeros((pad,), jnp.int32)]
    ).reshape(_E_PAD // _CH, _CH)
    dst = jnp.concatenate(
        [edge_index[1].astype(jnp.int32),
         jnp.full((pad,), N_PAD - 1, jnp.int32)]
    ).reshape(_E_PAD // _CH, _CH)
    batch2d = batch.astype(jnp.int32).reshape(N_NODES // _BM, _BM)

    y1 = _proj(x, W1a)
    p1 = _edge_agg(y1, src, dst)
    y2 = _mid(y1, p1, b1a.reshape(1, -1), W1b, b1b.reshape(1, -1), W2a)
    p2 = _edge_agg(y2, src, dst)
    return _tail_stage(y2, p2, b2a.reshape(1, -1), W2b, b2b.reshape(1, -1),
                       batch2d, Wg, bg.reshape(1, -1), Wo, bo.reshape(1, -1))


# speedup vs baseline: 2.7540x; 2.7540x over previous
import functools

import jax
import jax.numpy as jnp
from jax import lax
from jax.experimental import pallas as pl
from jax.experimental.pallas import tpu as pltpu
from jax.experimental.pallas import tpu_sc as plsc

N_NODES = 10000
N_PAD = 10240
N_EDGES = 320000
HID = 64
N_GRAPHS = 128

_NC, _NS = 2, 16
_NW = _NC * _NS
_CH = 128
_CPT = 80
_E_PAD = _NW * _CPT * _CH
_NBUF = 8
_LA = 4
_RPT = N_PAD // _NS


def _edge_agg(y, src2d, dst2d):
    mesh = plsc.VectorSubcoreMesh(core_axis_name="c", subcore_axis_name="s")

    @functools.partial(
        pl.kernel,
        mesh=mesh,
        compiler_params=pltpu.CompilerParams(use_tc_tiling_on_sc=False),
        out_type=jax.ShapeDtypeStruct((_NC, N_PAD, HID), jnp.float32),
        scratch_types=[
            pltpu.VMEM((_CPT, _CH), jnp.int32),
            pltpu.VMEM((_CPT, _CH), jnp.int32),
            [pltpu.VMEM((_CH, HID), jnp.float32) for _ in range(_NBUF)],
            pltpu.VMEM_SHARED((N_PAD, HID), jnp.float32),
            [pltpu.SemaphoreType.DMA for _ in range(_NBUF)],
            [pltpu.SemaphoreType.DMA for _ in range(_NBUF)],
        ],
    )
    def agg(y_hbm, src_hbm, dst_hbm, out_hbm,
            sidx, didx, rows, acc, sem_g, sem_s):
        c = lax.axis_index("c")
        s = lax.axis_index("s")
        tid = c * _NS + s

        zeros16 = jnp.zeros((16,), jnp.float32)
        zbuf = rows[0]

        def zrow(r, _):
            for j in range(HID // 16):
                zbuf[r, pl.ds(j * 16, 16)] = zeros16
            return ()

        lax.fori_loop(0, _CH, zrow, ())
        for i in range(_RPT // _CH):
            pltpu.sync_copy(zbuf, acc.at[pl.ds(s * _RPT + i * _CH, _CH)])

        r0 = tid * _CPT
        pltpu.sync_copy(src_hbm.at[pl.ds(r0, _CPT)], sidx)
        pltpu.sync_copy(dst_hbm.at[pl.ds(r0, _CPT)], didx)
        plsc.subcore_barrier()

        def g_start(j, b):
            pltpu.async_copy(y_hbm.at[sidx.at[j]], rows[b], sem_g[b])

        def g_wait(b):
            pltpu.make_async_copy(y_hbm.at[pl.ds(0, _CH)], rows[b],
                                  sem_g[b]).wait()

        def s_start(k, b):
            pltpu.async_copy(rows[b], acc.at[didx.at[k]], sem_s[b], add=True)

        def s_wait(b):
            pltpu.make_async_copy(rows[b], acc.at[pl.ds(0, _CH)],
                                  sem_s[b]).wait()

        for j in range(_LA):
            g_start(j, j)
        for k in range(_LA):
            g_start(k + _LA, k + _LA)
            g_wait(k)
            s_start(k, k)

        def group(g, _):
            for b8 in range(_NBUF):
                k = _LA + g * _NBUF + b8
                bj = b8
                s_wait(bj)
                g_start(k + _LA, bj)
                b = (_LA + b8) % _NBUF
                g_wait(b)
                s_start(k, b)
            return ()

        lax.fori_loop(0, (_CPT - 2 * _LA) // _NBUF, group, ())

        for k in range(_CPT - _LA, _CPT):
            b = k % _NBUF
            g_wait(b)
            s_start(k, b)
        for b in range(_NBUF):
            s_wait(b)

        plsc.subcore_barrier()
        pltpu.sync_copy(acc.at[pl.ds(s * _RPT, _RPT)],
                        out_hbm.at[c].at[pl.ds(s * _RPT, _RPT)])

    return agg(y, src2d, dst2d)


_BM = 1000


def _proj_body(x_ref, w_ref, o_ref):
    o_ref[...] = jnp.dot(x_ref[...], w_ref[...],
                         preferred_element_type=jnp.float32)


def _proj(x, w):
    m, k = x.shape
    n = w.shape[1]
    return pl.pallas_call(
        _proj_body,
        grid=(N_NODES // _BM,),
        in_specs=[
            pl.BlockSpec((_BM, k), lambda i: (i, 0)),
            pl.BlockSpec((k, n), lambda i: (0, 0)),
        ],
        out_specs=pl.BlockSpec((_BM, n), lambda i: (i, 0)),
        out_shape=jax.ShapeDtypeStruct((N_PAD, n), jnp.float32),
    )(x, w)


def _mid_body(y1_ref, p_ref, b1a_ref, w1b_ref, b1b_ref, w2a_ref, o_ref):
    u = jnp.maximum(y1_ref[...] + p_ref[0] + p_ref[1] + b1a_ref[...], 0.0)
    h = jnp.maximum(
        jnp.dot(u, w1b_ref[...], preferred_element_type=jnp.float32)
        + b1b_ref[...], 0.0)
    o_ref[...] = jnp.dot(h, w2a_ref[...], preferred_element_type=jnp.float32)


def _mid(y1, p, b1a, w1b, b1b, w2a):
    return pl.pallas_call(
        _mid_body,
        grid=(N_NODES // _BM,),
        in_specs=[
            pl.BlockSpec((_BM, HID), lambda i: (i, 0)),
            pl.BlockSpec((_NC, _BM, HID), lambda i: (0, i, 0)),
            pl.BlockSpec((1, HID), lambda i: (0, 0)),
            pl.BlockSpec((HID, HID), lambda i: (0, 0)),
            pl.BlockSpec((1, HID), lambda i: (0, 0)),
            pl.BlockSpec((HID, HID), lambda i: (0, 0)),
        ],
        out_specs=pl.BlockSpec((_BM, HID), lambda i: (i, 0)),
        out_shape=jax.ShapeDtypeStruct((N_PAD, HID), jnp.float32),
    )(y1, p, b1a, w1b, b1b, w2a)


def _tail_body(y2_ref, p_ref, b2a_ref, w2b_ref, b2b_ref, batch_ref,
               wg_ref, bg_ref, wo_ref, bo_ref, g_ref, o_ref):
    i = pl.program_id(0)
    ng = pl.num_programs(0)
    v = jnp.maximum(y2_ref[...] + p_ref[0] + p_ref[1] + b2a_ref[...], 0.0)
    h2 = jnp.maximum(
        jnp.dot(v, w2b_ref[...], preferred_element_type=jnp.float32)
        + b2b_ref[...], 0.0)
    ids = batch_ref[pl.ds(i, 1), :]
    onehot_t = (jnp.broadcast_to(ids, (N_GRAPHS, _BM))
                == lax.broadcasted_iota(jnp.int32, (N_GRAPHS, _BM), 0)
                ).astype(jnp.float32)
    gpart = lax.dot_general(onehot_t, h2, (((1,), (0,)), ((), ())),
                            preferred_element_type=jnp.float32)

    @pl.when(i == 0)
    def _init():
        g_ref[...] = gpart

    @pl.when(i > 0)
    def _accum():
        g_ref[...] += gpart

    @pl.when(i == ng - 1)
    def _head():
        g = g_ref[...]
        t = jnp.maximum(
            jnp.dot(g, wg_ref[...], preferred_element_type=jnp.float32)
            + bg_ref[...], 0.0)
        o_ref[...] = (jnp.dot(t, wo_ref[...],
                              preferred_element_type=jnp.float32)
                      + bo_ref[...])


def _tail_stage(y2, p, b2a, w2b, b2b, batch2d, wg, bg, wo, bo):
    out_dim = wo.shape[1]
    nb = N_NODES // _BM
    _, out = pl.pallas_call(
        _tail_body,
        grid=(nb,),
        in_specs=[
            pl.BlockSpec((_BM, HID), lambda i: (i, 0)),
            pl.BlockSpec((_NC, _BM, HID), lambda i: (0, i, 0)),
            pl.BlockSpec((1, HID), lambda i: (0, 0)),
            pl.BlockSpec((HID, HID), lambda i: (0, 0)),
            pl.BlockSpec((1, HID), lambda i: (0, 0)),
            pl.BlockSpec((nb, _BM), lambda i: (0, 0)),
            pl.BlockSpec((HID, HID), lambda i: (0, 0)),
            pl.BlockSpec((1, HID), lambda i: (0, 0)),
            pl.BlockSpec((HID, out_dim), lambda i: (0, 0)),
            pl.BlockSpec((1, out_dim), lambda i: (0, 0)),
        ],
        out_specs=[
            pl.BlockSpec((N_GRAPHS, HID), lambda i: (0, 0)),
            pl.BlockSpec((N_GRAPHS, out_dim), lambda i: (0, 0)),
        ],
        out_shape=[
            jax.ShapeDtypeStruct((N_GRAPHS, HID), jnp.float32),
            jax.ShapeDtypeStruct((N_GRAPHS, out_dim), jnp.float32),
        ],
    )(y2, p, b2a, w2b, b2b, batch2d, wg, bg, wo, bo)
    return out


def kernel(x, edge_index, batch, W1a, b1a, W1b, b1b, W2a, b2a, W2b, b2b,
           Wg, bg, Wo, bo):
    pad = _E_PAD - N_EDGES
    pad_ar = jnp.arange(pad, dtype=jnp.int32)
    src = jnp.concatenate(
        [edge_index[0].astype(jnp.int32), pad_ar % N_NODES]
    ).reshape(_E_PAD // _CH, _CH)
    dst = jnp.concatenate(
        [edge_index[1].astype(jnp.int32),
         N_NODES + pad_ar % (N_PAD - N_NODES)]
    ).reshape(_E_PAD // _CH, _CH)
    batch2d = batch.astype(jnp.int32).reshape(N_NODES // _BM, _BM)

    y1 = _proj(x, W1a)
    p1 = _edge_agg(y1, src, dst)
    y2 = _mid(y1, p1, b1a.reshape(1, -1), W1b, b1b.reshape(1, -1), W2a)
    p2 = _edge_agg(y2, src, dst)
    return _tail_stage(y2, p2, b2a.reshape(1, -1), W2b, b2b.reshape(1, -1),
                       batch2d, Wg, bg.reshape(1, -1), Wo, bo.reshape(1, -1))


# speedup vs baseline: 2.8726x; 1.0431x over previous
import functools

import jax
import jax.numpy as jnp
from jax import lax
from jax.experimental import pallas as pl
from jax.experimental.pallas import tpu as pltpu
from jax.experimental.pallas import tpu_sc as plsc

N_NODES = 10000
N_PAD = 10240
N_EDGES = 320000
HID = 64
N_GRAPHS = 128

_NC, _NS = 2, 16
_NW = _NC * _NS
_CH = 128
_CPT = 80
_E_PAD = _NW * _CPT * _CH
_NBUF = 8
_LA = 4
_RPT = N_PAD // _NS


def _edge_agg(y, src2d, dst2d):
    mesh = plsc.VectorSubcoreMesh(core_axis_name="c", subcore_axis_name="s")

    @functools.partial(
        pl.kernel,
        mesh=mesh,
        compiler_params=pltpu.CompilerParams(use_tc_tiling_on_sc=False),
        out_type=jax.ShapeDtypeStruct((_NC, N_PAD, HID), jnp.float32),
        scratch_types=[
            pltpu.VMEM((_CPT, _CH), jnp.int32),
            pltpu.VMEM((_CPT, _CH), jnp.int32),
            [pltpu.VMEM((_CH, HID), jnp.float32) for _ in range(_NBUF)],
            pltpu.VMEM_SHARED((N_PAD, HID), jnp.float32),
            [pltpu.SemaphoreType.DMA for _ in range(_NBUF)],
            [pltpu.SemaphoreType.DMA for _ in range(_NBUF)],
        ],
    )
    def agg(y_hbm, src_hbm, dst_hbm, out_hbm,
            sidx, didx, rows, acc, sem_g, sem_s):
        c = lax.axis_index("c")
        s = lax.axis_index("s")
        tid = c * _NS + s

        zeros16 = jnp.zeros((16,), jnp.float32)
        zbuf = rows[0]

        def zrow(r, _):
            for j in range(HID // 16):
                zbuf[r, pl.ds(j * 16, 16)] = zeros16
            return ()

        lax.fori_loop(0, _CH, zrow, ())
        for i in range(_RPT // _CH):
            pltpu.sync_copy(zbuf, acc.at[pl.ds(s * _RPT + i * _CH, _CH)])

        r0 = tid * _CPT
        pltpu.sync_copy(src_hbm.at[pl.ds(r0, _CPT)], sidx)
        pltpu.sync_copy(dst_hbm.at[pl.ds(r0, _CPT)], didx)
        plsc.subcore_barrier()

        def g_start(j, b):
            pltpu.async_copy(y_hbm.at[sidx.at[j]], rows[b], sem_g[b])

        def g_wait(b):
            pltpu.make_async_copy(y_hbm.at[pl.ds(0, _CH)], rows[b],
                                  sem_g[b]).wait()

        def s_start(k, b):
            pltpu.async_copy(rows[b], acc.at[didx.at[k]], sem_s[b], add=True)

        def s_wait(b):
            pltpu.make_async_copy(rows[b], acc.at[pl.ds(0, _CH)],
                                  sem_s[b]).wait()

        for j in range(_LA):
            g_start(j, j)
        for k in range(_LA):
            g_start(k + _LA, k + _LA)
            g_wait(k)
            s_start(k, k)

        def group(g, _):
            for b8 in range(_NBUF):
                k = _LA + g * _NBUF + b8
                bj = b8
                s_wait(bj)
                g_start(k + _LA, bj)
                b = (_LA + b8) % _NBUF
                g_wait(b)
                s_start(k, b)
            return ()

        lax.fori_loop(0, (_CPT - 2 * _LA) // _NBUF, group, ())

        for k in range(_CPT - _LA, _CPT):
            b = k % _NBUF
            g_wait(b)
            s_start(k, b)
        for b in range(_NBUF):
            s_wait(b)

        plsc.subcore_barrier()
        pltpu.sync_copy(acc.at[pl.ds(s * _RPT, _RPT)],
                        out_hbm.at[c].at[pl.ds(s * _RPT, _RPT)])

    return agg(y, src2d, dst2d)


_BM = 2000


def _proj_body(x_ref, w_ref, o_ref):
    o_ref[...] = jnp.dot(x_ref[...], w_ref[...],
                         preferred_element_type=jnp.float32)


def _proj(x, w):
    m, k = x.shape
    n = w.shape[1]
    return pl.pallas_call(
        _proj_body,
        grid=(N_NODES // _BM,),
        in_specs=[
            pl.BlockSpec((_BM, k), lambda i: (i, 0)),
            pl.BlockSpec((k, n), lambda i: (0, 0)),
        ],
        out_specs=pl.BlockSpec((_BM, n), lambda i: (i, 0)),
        out_shape=jax.ShapeDtypeStruct((N_PAD, n), jnp.float32),
    )(x, w)


def _mid_body(y1_ref, p_ref, b1a_ref, w1b_ref, b1b_ref, w2a_ref, o_ref):
    u = jnp.maximum(y1_ref[...] + p_ref[0] + p_ref[1] + b1a_ref[...], 0.0)
    h = jnp.maximum(
        jnp.dot(u, w1b_ref[...], preferred_element_type=jnp.float32)
        + b1b_ref[...], 0.0)
    o_ref[...] = jnp.dot(h, w2a_ref[...], preferred_element_type=jnp.float32)


def _mid(y1, p, b1a, w1b, b1b, w2a):
    return pl.pallas_call(
        _mid_body,
        grid=(N_NODES // _BM,),
        in_specs=[
            pl.BlockSpec((_BM, HID), lambda i: (i, 0)),
            pl.BlockSpec((_NC, _BM, HID), lambda i: (0, i, 0)),
            pl.BlockSpec((1, HID), lambda i: (0, 0)),
            pl.BlockSpec((HID, HID), lambda i: (0, 0)),
            pl.BlockSpec((1, HID), lambda i: (0, 0)),
            pl.BlockSpec((HID, HID), lambda i: (0, 0)),
        ],
        out_specs=pl.BlockSpec((_BM, HID), lambda i: (i, 0)),
        out_shape=jax.ShapeDtypeStruct((N_PAD, HID), jnp.float32),
    )(y1, p, b1a, w1b, b1b, w2a)


def _tail_body(y2_ref, p_ref, b2a_ref, w2b_ref, b2b_ref, batch_ref,
               wg_ref, bg_ref, wo_ref, bo_ref, g_ref, o_ref):
    i = pl.program_id(0)
    ng = pl.num_programs(0)
    v = jnp.maximum(y2_ref[...] + p_ref[0] + p_ref[1] + b2a_ref[...], 0.0)
    h2 = jnp.maximum(
        jnp.dot(v, w2b_ref[...], preferred_element_type=jnp.float32)
        + b2b_ref[...], 0.0)
    ids = batch_ref[pl.ds(i, 1), :]
    onehot_t = (jnp.broadcast_to(ids, (N_GRAPHS, _BM))
                == lax.broadcasted_iota(jnp.int32, (N_GRAPHS, _BM), 0)
                ).astype(jnp.float32)
    gpart = lax.dot_general(onehot_t, h2, (((1,), (0,)), ((), ())),
                            preferred_element_type=jnp.float32)

    @pl.when(i == 0)
    def _init():
        g_ref[...] = gpart

    @pl.when(i > 0)
    def _accum():
        g_ref[...] += gpart

    @pl.when(i == ng - 1)
    def _head():
        g = g_ref[...]
        t = jnp.maximum(
            jnp.dot(g, wg_ref[...], preferred_element_type=jnp.float32)
            + bg_ref[...], 0.0)
        o_ref[...] = (jnp.dot(t, wo_ref[...],
                              preferred_element_type=jnp.float32)
                      + bo_ref[...])


def _tail_stage(y2, p, b2a, w2b, b2b, batch2d, wg, bg, wo, bo):
    out_dim = wo.shape[1]
    nb = N_NODES // _BM
    _, out = pl.pallas_call(
        _tail_body,
        grid=(nb,),
        in_specs=[
            pl.BlockSpec((_BM, HID), lambda i: (i, 0)),
            pl.BlockSpec((_NC, _BM, HID), lambda i: (0, i, 0)),
            pl.BlockSpec((1, HID), lambda i: (0, 0)),
            pl.BlockSpec((HID, HID), lambda i: (0, 0)),
            pl.BlockSpec((1, HID), lambda i: (0, 0)),
            pl.BlockSpec((nb, _BM), lambda i: (0, 0)),
            pl.BlockSpec((HID, HID), lambda i: (0, 0)),
            pl.BlockSpec((1, HID), lambda i: (0, 0)),
            pl.BlockSpec((HID, out_dim), lambda i: (0, 0)),
            pl.BlockSpec((1, out_dim), lambda i: (0, 0)),
        ],
        out_specs=[
            pl.BlockSpec((N_GRAPHS, HID), lambda i: (0, 0)),
            pl.BlockSpec((N_GRAPHS, out_dim), lambda i: (0, 0)),
        ],
        out_shape=[
            jax.ShapeDtypeStruct((N_GRAPHS, HID), jnp.float32),
            jax.ShapeDtypeStruct((N_GRAPHS, out_dim), jnp.float32),
        ],
    )(y2, p, b2a, w2b, b2b, batch2d, wg, bg, wo, bo)
    return out


def kernel(x, edge_index, batch, W1a, b1a, W1b, b1b, W2a, b2a, W2b, b2b,
           Wg, bg, Wo, bo):
    pad = _E_PAD - N_EDGES
    pad_ar = jnp.arange(pad, dtype=jnp.int32)
    src = jnp.concatenate(
        [edge_index[0].astype(jnp.int32), pad_ar % N_NODES]
    ).reshape(_E_PAD // _CH, _CH)
    dst = jnp.concatenate(
        [edge_index[1].astype(jnp.int32),
         N_NODES + pad_ar % (N_PAD - N_NODES)]
    ).reshape(_E_PAD // _CH, _CH)
    batch2d = batch.astype(jnp.int32).reshape(N_NODES // _BM, _BM)

    y1 = _proj(x, W1a)
    p1 = _edge_agg(y1, src, dst)
    y2 = _mid(y1, p1, b1a.reshape(1, -1), W1b, b1b.reshape(1, -1), W2a)
    p2 = _edge_agg(y2, src, dst)
    return _tail_stage(y2, p2, b2a.reshape(1, -1), W2b, b2b.reshape(1, -1),
                       batch2d, Wg, bg.reshape(1, -1), Wo, bo.reshape(1, -1))


# speedup vs baseline: 3.4652x; 1.2063x over previous
import functools

import jax
import jax.numpy as jnp
from jax import lax
from jax.experimental import pallas as pl
from jax.experimental.pallas import tpu as pltpu
from jax.experimental.pallas import tpu_sc as plsc

N_NODES = 10000
N_PAD = 10240
N_EDGES = 320000
HID = 64
N_GRAPHS = 128

_NC, _NS = 2, 16
_NW = _NC * _NS
_CH = 128
_CPT = 80
_E_PAD = _NW * _CPT * _CH
_NBUF = 8
_LA = 4
_RPT = N_PAD // _NS


def _edge_agg(y, src2d, dst2d):
    mesh = plsc.VectorSubcoreMesh(core_axis_name="c", subcore_axis_name="s")

    @functools.partial(
        pl.kernel,
        mesh=mesh,
        compiler_params=pltpu.CompilerParams(use_tc_tiling_on_sc=False),
        out_type=[jax.ShapeDtypeStruct((N_PAD, HID), jnp.float32),
                  jax.ShapeDtypeStruct((N_PAD, HID), jnp.float32)],
        scratch_types=[
            pltpu.VMEM((_CPT, _CH), jnp.int32),
            pltpu.VMEM((_CPT, _CH), jnp.int32),
            [pltpu.VMEM((_CH, HID), jnp.float32) for _ in range(_NBUF)],
            pltpu.VMEM_SHARED((N_PAD, HID), jnp.float32),
            [pltpu.SemaphoreType.DMA for _ in range(_NBUF)],
            [pltpu.SemaphoreType.DMA for _ in range(_NBUF)],
        ],
    )
    def agg(y_hbm, src_hbm, dst_hbm, out0_hbm, out1_hbm,
            sidx, didx, rows, acc, sem_g, sem_s):
        c = lax.axis_index("c")
        s = lax.axis_index("s")
        tid = c * _NS + s

        zeros16 = jnp.zeros((16,), jnp.float32)
        zbuf = rows[0]

        def zrow(r, _):
            for j in range(HID // 16):
                zbuf[r, pl.ds(j * 16, 16)] = zeros16
            return ()

        lax.fori_loop(0, _CH, zrow, ())
        for i in range(_RPT // _CH):
            pltpu.sync_copy(zbuf, acc.at[pl.ds(s * _RPT + i * _CH, _CH)])

        r0 = tid * _CPT
        pltpu.sync_copy(src_hbm.at[pl.ds(r0, _CPT)], sidx)
        pltpu.sync_copy(dst_hbm.at[pl.ds(r0, _CPT)], didx)
        plsc.subcore_barrier()

        def g_start(j, b):
            pltpu.async_copy(y_hbm.at[sidx.at[j]], rows[b], sem_g[b])

        def g_wait(b):
            pltpu.make_async_copy(y_hbm.at[pl.ds(0, _CH)], rows[b],
                                  sem_g[b]).wait()

        def s_start(k, b):
            pltpu.async_copy(rows[b], acc.at[didx.at[k]], sem_s[b], add=True)

        def s_wait(b):
            pltpu.make_async_copy(rows[b], acc.at[pl.ds(0, _CH)],
                                  sem_s[b]).wait()

        for j in range(_LA):
            g_start(j, j)
        for k in range(_LA):
            g_start(k + _LA, k + _LA)
            g_wait(k)
            s_start(k, k)

        def group(g, _):
            for b8 in range(_NBUF):
                k = _LA + g * _NBUF + b8
                bj = b8
                s_wait(bj)
                g_start(k + _LA, bj)
                b = (_LA + b8) % _NBUF
                g_wait(b)
                s_start(k, b)
            return ()

        lax.fori_loop(0, (_CPT - 2 * _LA) // _NBUF, group, ())

        for k in range(_CPT - _LA, _CPT):
            b = k % _NBUF
            g_wait(b)
            s_start(k, b)
        for b in range(_NBUF):
            s_wait(b)

        plsc.subcore_barrier()

        @pl.when(c == 0)
        def _flush0():
            pltpu.sync_copy(acc.at[pl.ds(s * _RPT, _RPT)],
                            out0_hbm.at[pl.ds(s * _RPT, _RPT)])

        @pl.when(c == 1)
        def _flush1():
            pltpu.sync_copy(acc.at[pl.ds(s * _RPT, _RPT)],
                            out1_hbm.at[pl.ds(s * _RPT, _RPT)])

    return agg(y, src2d, dst2d)


_BM = 1024
_HN = N_PAD // 2


def _proj_body(xa_ref, xb_ref, w_ref, o_ref):
    i = pl.program_id(0)
    a = jnp.dot(xa_ref[...], w_ref[...], preferred_element_type=jnp.float32)
    b = jnp.dot(xb_ref[...], w_ref[...], preferred_element_type=jnp.float32)
    nvalid_b = N_NODES - _HN - i * _BM
    valid_b = lax.broadcasted_iota(jnp.int32, (_BM, HID), 0) < nvalid_b
    o_ref[...] = jnp.concatenate([a, jnp.where(valid_b, b, 0.0)], axis=1)


def _proj(x, w):
    k = x.shape[1]
    n = w.shape[1]
    nb = _HN // _BM
    return pl.pallas_call(
        _proj_body,
        grid=(nb,),
        in_specs=[
            pl.BlockSpec((_BM, k), lambda i: (i, 0)),
            pl.BlockSpec((_BM, k), lambda i: (i + _HN // _BM, 0)),
            pl.BlockSpec((k, n), lambda i: (0, 0)),
        ],
        out_specs=pl.BlockSpec((_BM, 2 * n), lambda i: (i, 0)),
        out_shape=jax.ShapeDtypeStruct((_HN, 2 * n), jnp.float32),
    )(x, x, w)


def _mid_body(y1_ref, p0_ref, p1_ref, b1a_ref, w1b_ref, b1b_ref, w2a_ref,
              o_ref):
    u = jnp.maximum(y1_ref[...] + p0_ref[...] + p1_ref[...] + b1a_ref[...],
                    0.0)
    h = jnp.maximum(
        jnp.dot(u, w1b_ref[...], preferred_element_type=jnp.float32)
        + b1b_ref[...], 0.0)
    o_ref[...] = jnp.dot(h, w2a_ref[...], preferred_element_type=jnp.float32)


def _mid(y1, p0, p1, b1a2, w1b_bd, b1b2, w2a_bd):
    return pl.pallas_call(
        _mid_body,
        grid=(_HN // _BM,),
        in_specs=[
            pl.BlockSpec((_BM, 2 * HID), lambda i: (i, 0)),
            pl.BlockSpec((_BM, 2 * HID), lambda i: (i, 0)),
            pl.BlockSpec((_BM, 2 * HID), lambda i: (i, 0)),
            pl.BlockSpec((1, 2 * HID), lambda i: (0, 0)),
            pl.BlockSpec((2 * HID, 2 * HID), lambda i: (0, 0)),
            pl.BlockSpec((1, 2 * HID), lambda i: (0, 0)),
            pl.BlockSpec((2 * HID, 2 * HID), lambda i: (0, 0)),
        ],
        out_specs=pl.BlockSpec((_BM, 2 * HID), lambda i: (i, 0)),
        out_shape=jax.ShapeDtypeStruct((_HN, 2 * HID), jnp.float32),
    )(y1, p0, p1, b1a2, w1b_bd, b1b2, w2a_bd)


def _tail_body(y2_ref, p0_ref, p1_ref, b2a_ref, w2b_ref, b2b_ref, batch_ref,
               wg_ref, bg_ref, wo_ref, bo_ref, g_ref, o_ref):
    i = pl.program_id(0)
    ng = pl.num_programs(0)
    v = jnp.maximum(y2_ref[...] + p0_ref[...] + p1_ref[...] + b2a_ref[...],
                    0.0)
    h2 = jnp.maximum(
        jnp.dot(v, w2b_ref[...], preferred_element_type=jnp.float32)
        + b2b_ref[...], 0.0)
    gio = lax.broadcasted_iota(jnp.int32, (N_GRAPHS, _BM), 0)
    ids_a = batch_ref[pl.ds(i, 1), :]
    ids_b = batch_ref[pl.ds(i + ng, 1), :]
    onehot_a = (jnp.broadcast_to(ids_a, (N_GRAPHS, _BM)) == gio
                ).astype(jnp.float32)
    onehot_b = (jnp.broadcast_to(ids_b, (N_GRAPHS, _BM)) == gio
                ).astype(jnp.float32)
    nvalid_b = N_NODES - _HN - i * _BM
    valid_b = lax.broadcasted_iota(jnp.int32, (_BM, HID), 0) < nvalid_b
    h2b = jnp.where(valid_b, h2[:, HID:], 0.0)
    gpart = (lax.dot_general(onehot_a, h2[:, :HID], (((1,), (0,)), ((), ())),
                             preferred_element_type=jnp.float32)
             + lax.dot_general(onehot_b, h2b, (((1,), (0,)), ((), ())),
                               preferred_element_type=jnp.float32))

    @pl.when(i == 0)
    def _init():
        g_ref[...] = gpart

    @pl.when(i > 0)
    def _accum():
        g_ref[...] += gpart

    @pl.when(i == ng - 1)
    def _head():
        g = g_ref[...]
        t = jnp.maximum(
            jnp.dot(g, wg_ref[...], preferred_element_type=jnp.float32)
            + bg_ref[...], 0.0)
        o_ref[...] = (jnp.dot(t, wo_ref[...],
                              preferred_element_type=jnp.float32)
                      + bo_ref[...])


def _tail_stage(y2, p0, p1, b2a2, w2b_bd, b2b2, batch2d, wg2, bg, wo, bo):
    out_dim = wo.shape[1]
    nb = _HN // _BM
    _, out = pl.pallas_call(
        _tail_body,
        grid=(nb,),
        in_specs=[
            pl.BlockSpec((_BM, 2 * HID), lambda i: (i, 0)),
            pl.BlockSpec((_BM, 2 * HID), lambda i: (i, 0)),
            pl.BlockSpec((_BM, 2 * HID), lambda i: (i, 0)),
            pl.BlockSpec((1, 2 * HID), lambda i: (0, 0)),
            pl.BlockSpec((2 * HID, 2 * HID), lambda i: (0, 0)),
            pl.BlockSpec((1, 2 * HID), lambda i: (0, 0)),
            pl.BlockSpec((2 * nb, _BM), lambda i: (0, 0)),
            pl.BlockSpec((HID, HID), lambda i: (0, 0)),
            pl.BlockSpec((1, HID), lambda i: (0, 0)),
            pl.BlockSpec((HID, out_dim), lambda i: (0, 0)),
            pl.BlockSpec((1, out_dim), lambda i: (0, 0)),
        ],
        out_specs=[
            pl.BlockSpec((N_GRAPHS, HID), lambda i: (0, 0)),
            pl.BlockSpec((N_GRAPHS, out_dim), lambda i: (0, 0)),
        ],
        out_shape=[
            jax.ShapeDtypeStruct((N_GRAPHS, HID), jnp.float32),
            jax.ShapeDtypeStruct((N_GRAPHS, out_dim), jnp.float32),
        ],
    )(y2, p0, p1, b2a2, w2b_bd, b2b2, batch2d, wg2, bg, wo, bo)
    return out


def _pair2(v):
    return jnp.concatenate([v, v]).reshape(1, -1)


def _blockdiag(w):
    z = jnp.zeros_like(w)
    return jnp.concatenate(
        [jnp.concatenate([w, z], axis=1), jnp.concatenate([z, w], axis=1)],
        axis=0)


def _sigma(i):
    return jnp.where(i < _HN, 2 * i, 2 * i - (N_PAD - 1))


def kernel(x, edge_index, batch, W1a, b1a, W1b, b1b, W2a, b2a, W2b, b2b,
           Wg, bg, Wo, bo):
    pad = _E_PAD - N_EDGES
    pad_ar = jnp.arange(pad, dtype=jnp.int32)
    src = _sigma(jnp.concatenate(
        [edge_index[0].astype(jnp.int32), pad_ar % N_NODES]
    )).reshape(_E_PAD // _CH, _CH)
    dst = _sigma(jnp.concatenate(
        [edge_index[1].astype(jnp.int32),
         N_NODES + pad_ar % (N_PAD - N_NODES)]
    )).reshape(_E_PAD // _CH, _CH)
    nb = _HN // _BM
    batch2d = jnp.concatenate(
        [batch.astype(jnp.int32),
         jnp.zeros((N_PAD - N_NODES,), jnp.int32)]).reshape(2 * nb, _BM)

    y1 = _proj(x, W1a)
    p1a, p1b = _edge_agg(y1.reshape(N_PAD, HID), src, dst)
    y2 = _mid(y1, p1a.reshape(_HN, 2 * HID), p1b.reshape(_HN, 2 * HID),
              _pair2(b1a), _blockdiag(W1b), _pair2(b1b), _blockdiag(W2a))
    p2a, p2b = _edge_agg(y2.reshape(N_PAD, HID), src, dst)
    return _tail_stage(y2, p2a.reshape(_HN, 2 * HID), p2b.reshape(_HN, 2 * HID),
                       _pair2(b2a), _blockdiag(W2b), _pair2(b2b),
                       batch2d, Wg, bg.reshape(1, -1), Wo, bo.reshape(1, -1))


# speedup vs baseline: 3.5616x; 1.0278x over previous
import functools

import jax
import jax.numpy as jnp
from jax import lax
from jax.experimental import pallas as pl
from jax.experimental.pallas import tpu as pltpu
from jax.experimental.pallas import tpu_sc as plsc

N_NODES = 10000
N_PAD = 10240
N_EDGES = 320000
HID = 64
N_GRAPHS = 128

_NC, _NS = 2, 16
_NW = _NC * _NS
_CH = 128
_CPT = 80
_E_PAD = _NW * _CPT * _CH
_NBUF = 8
_LA = 4
_RPT = N_PAD // _NS


def _edge_agg(y, src2d, dst2d):
    mesh = plsc.VectorSubcoreMesh(core_axis_name="c", subcore_axis_name="s")

    @functools.partial(
        pl.kernel,
        mesh=mesh,
        compiler_params=pltpu.CompilerParams(use_tc_tiling_on_sc=False),
        out_type=[jax.ShapeDtypeStruct((N_PAD, HID), jnp.float32),
                  jax.ShapeDtypeStruct((N_PAD, HID), jnp.float32)],
        scratch_types=[
            pltpu.VMEM((_CPT, _CH), jnp.int32),
            pltpu.VMEM((_CPT, _CH), jnp.int32),
            [pltpu.VMEM((_CH, HID), jnp.float32) for _ in range(_NBUF)],
            pltpu.VMEM_SHARED((N_PAD, HID), jnp.float32),
            [pltpu.SemaphoreType.DMA for _ in range(_NBUF)],
            [pltpu.SemaphoreType.DMA for _ in range(_NBUF)],
        ],
    )
    def agg(y_hbm, src_hbm, dst_hbm, out0_hbm, out1_hbm,
            sidx, didx, rows, acc, sem_g, sem_s):
        c = lax.axis_index("c")
        s = lax.axis_index("s")
        tid = c * _NS + s

        zeros16 = jnp.zeros((16,), jnp.float32)
        zbuf = rows[0]

        def zrow(r, _):
            for j in range(HID // 16):
                zbuf[r, pl.ds(j * 16, 16)] = zeros16
            return ()

        lax.fori_loop(0, _CH, zrow, ())
        for i in range(_RPT // _CH):
            pltpu.sync_copy(zbuf, acc.at[pl.ds(s * _RPT + i * _CH, _CH)])

        r0 = tid * _CPT
        pltpu.sync_copy(src_hbm.at[pl.ds(r0, _CPT)], sidx)
        pltpu.sync_copy(dst_hbm.at[pl.ds(r0, _CPT)], didx)
        plsc.subcore_barrier()

        def g_start(j, b):
            pltpu.async_copy(y_hbm.at[sidx.at[j]], rows[b], sem_g[b])

        def g_wait(b):
            pltpu.make_async_copy(y_hbm.at[pl.ds(0, _CH)], rows[b],
                                  sem_g[b]).wait()

        def s_start(k, b):
            pltpu.async_copy(rows[b], acc.at[didx.at[k]], sem_s[b], add=True)

        def s_wait(b):
            pltpu.make_async_copy(rows[b], acc.at[pl.ds(0, _CH)],
                                  sem_s[b]).wait()

        for j in range(_LA):
            g_start(j, j)
        for k in range(_LA):
            g_start(k + _LA, k + _LA)
            g_wait(k)
            s_start(k, k)

        def group(g, _):
            for b8 in range(_NBUF):
                k = _LA + g * _NBUF + b8
                bj = b8
                s_wait(bj)
                g_start(k + _LA, bj)
                b = (_LA + b8) % _NBUF
                g_wait(b)
                s_start(k, b)
            return ()

        lax.fori_loop(0, (_CPT - 2 * _LA) // _NBUF, group, ())

        for k in range(_CPT - _LA, _CPT):
            b = k % _NBUF
            g_wait(b)
            s_start(k, b)
        for b in range(_NBUF):
            s_wait(b)

        plsc.subcore_barrier()

        @pl.when(c == 0)
        def _flush0():
            pltpu.sync_copy(acc.at[pl.ds(s * _RPT, _RPT)],
                            out0_hbm.at[pl.ds(s * _RPT, _RPT)])

        @pl.when(c == 1)
        def _flush1():
            pltpu.sync_copy(acc.at[pl.ds(s * _RPT, _RPT)],
                            out1_hbm.at[pl.ds(s * _RPT, _RPT)])

    return agg(y, src2d, dst2d)


_BM = 1024
_HN = N_PAD // 2


def _proj_body(xa_ref, xb_ref, w_ref, o_ref):
    i = pl.program_id(0)
    a = jnp.dot(xa_ref[...], w_ref[...], preferred_element_type=jnp.float32)
    b = jnp.dot(xb_ref[...], w_ref[...], preferred_element_type=jnp.float32)
    nvalid_b = N_NODES - _HN - i * _BM
    valid_b = lax.broadcasted_iota(jnp.int32, (_BM, HID), 0) < nvalid_b
    o_ref[...] = jnp.concatenate([a, jnp.where(valid_b, b, 0.0)], axis=1)


def _proj(x, w):
    k = x.shape[1]
    n = w.shape[1]
    nb = _HN // _BM
    return pl.pallas_call(
        _proj_body,
        grid=(nb,),
        in_specs=[
            pl.BlockSpec((_BM, k), lambda i: (i, 0)),
            pl.BlockSpec((_BM, k), lambda i: (i + _HN // _BM, 0)),
            pl.BlockSpec((k, n), lambda i: (0, 0)),
        ],
        out_specs=pl.BlockSpec((_BM, 2 * n), lambda i: (i, 0)),
        out_shape=jax.ShapeDtypeStruct((_HN, 2 * n), jnp.float32),
    )(x, x, w)


def _mid_body(y1_ref, p0_ref, p1_ref, b1a_ref, w1b_ref, b1b_ref, w2a_ref,
              o_ref):
    u = jnp.maximum(y1_ref[...] + p0_ref[...] + p1_ref[...] + b1a_ref[...],
                    0.0)
    h = jnp.maximum(
        jnp.dot(u, w1b_ref[...], preferred_element_type=jnp.float32)
        + b1b_ref[...], 0.0)
    o_ref[...] = jnp.dot(h, w2a_ref[...], preferred_element_type=jnp.float32)


def _mid(y1, p0, p1, b1a2, w1b_bd, b1b2, w2a_bd):
    return pl.pallas_call(
        _mid_body,
        grid=(_HN // _BM,),
        in_specs=[
            pl.BlockSpec((_BM, 2 * HID), lambda i: (i, 0)),
            pl.BlockSpec((_BM, 2 * HID), lambda i: (i, 0)),
            pl.BlockSpec((_BM, 2 * HID), lambda i: (i, 0)),
            pl.BlockSpec((1, 2 * HID), lambda i: (0, 0)),
            pl.BlockSpec((2 * HID, 2 * HID), lambda i: (0, 0)),
            pl.BlockSpec((1, 2 * HID), lambda i: (0, 0)),
            pl.BlockSpec((2 * HID, 2 * HID), lambda i: (0, 0)),
        ],
        out_specs=pl.BlockSpec((_BM, 2 * HID), lambda i: (i, 0)),
        out_shape=jax.ShapeDtypeStruct((_HN, 2 * HID), jnp.float32),
    )(y1, p0, p1, b1a2, w1b_bd, b1b2, w2a_bd)


def _tail_body(y2_ref, p0_ref, p1_ref, b2a_ref, w2b_ref, b2b_ref, batch_ref,
               wg_ref, bg_ref, wo_ref, bo_ref, g_ref, o_ref):
    i = pl.program_id(0)
    ng = pl.num_programs(0)
    v = jnp.maximum(y2_ref[...] + p0_ref[...] + p1_ref[...] + b2a_ref[...],
                    0.0)
    h2 = jnp.maximum(
        jnp.dot(v, w2b_ref[...], preferred_element_type=jnp.float32)
        + b2b_ref[...], 0.0)
    gio = lax.broadcasted_iota(jnp.int32, (N_GRAPHS, _BM), 0)
    ids_a = batch_ref[pl.ds(i, 1), :]
    ids_b = batch_ref[pl.ds(i + ng, 1), :]
    onehot_a = (jnp.broadcast_to(ids_a, (N_GRAPHS, _BM)) == gio
                ).astype(jnp.float32)
    onehot_b = (jnp.broadcast_to(ids_b, (N_GRAPHS, _BM)) == gio
                ).astype(jnp.float32)
    nvalid_b = N_NODES - _HN - i * _BM
    valid_b = lax.broadcasted_iota(jnp.int32, (_BM, HID), 0) < nvalid_b
    h2b = jnp.where(valid_b, h2[:, HID:], 0.0)
    gpart = (lax.dot_general(onehot_a, h2[:, :HID], (((1,), (0,)), ((), ())),
                             preferred_element_type=jnp.float32)
             + lax.dot_general(onehot_b, h2b, (((1,), (0,)), ((), ())),
                               preferred_element_type=jnp.float32))

    @pl.when(i == 0)
    def _init():
        g_ref[...] = gpart

    @pl.when(i > 0)
    def _accum():
        g_ref[...] += gpart

    @pl.when(i == ng - 1)
    def _head():
        g = g_ref[...]
        t = jnp.maximum(
            jnp.dot(g, wg_ref[...], preferred_element_type=jnp.float32)
            + bg_ref[...], 0.0)
        o_ref[...] = (jnp.dot(t, wo_ref[...],
                              preferred_element_type=jnp.float32)
                      + bo_ref[...])


def _tail_stage(y2, p0, p1, b2a2, w2b_bd, b2b2, batch2d, wg2, bg, wo, bo):
    out_dim = wo.shape[1]
    nb = _HN // _BM
    _, out = pl.pallas_call(
        _tail_body,
        grid=(nb,),
        in_specs=[
            pl.BlockSpec((_BM, 2 * HID), lambda i: (i, 0)),
            pl.BlockSpec((_BM, 2 * HID), lambda i: (i, 0)),
            pl.BlockSpec((_BM, 2 * HID), lambda i: (i, 0)),
            pl.BlockSpec((1, 2 * HID), lambda i: (0, 0)),
            pl.BlockSpec((2 * HID, 2 * HID), lambda i: (0, 0)),
            pl.BlockSpec((1, 2 * HID), lambda i: (0, 0)),
            pl.BlockSpec((2 * nb, _BM), lambda i: (0, 0)),
            pl.BlockSpec((HID, HID), lambda i: (0, 0)),
            pl.BlockSpec((1, HID), lambda i: (0, 0)),
            pl.BlockSpec((HID, out_dim), lambda i: (0, 0)),
            pl.BlockSpec((1, out_dim), lambda i: (0, 0)),
        ],
        out_specs=[
            pl.BlockSpec((N_GRAPHS, HID), lambda i: (0, 0)),
            pl.BlockSpec((N_GRAPHS, out_dim), lambda i: (0, 0)),
        ],
        out_shape=[
            jax.ShapeDtypeStruct((N_GRAPHS, HID), jnp.float32),
            jax.ShapeDtypeStruct((N_GRAPHS, out_dim), jnp.float32),
        ],
    )(y2, p0, p1, b2a2, w2b_bd, b2b2, batch2d, wg2, bg, wo, bo)
    return out


def _pair2(v):
    return jnp.concatenate([v, v]).reshape(1, -1)


def _blockdiag(w):
    z = jnp.zeros_like(w)
    return jnp.concatenate(
        [jnp.concatenate([w, z], axis=1), jnp.concatenate([z, w], axis=1)],
        axis=0)


def _sigma(i):
    return jnp.where(i < _HN, 2 * i, 2 * i - (N_PAD - 1))


_EB = N_EDGES // 5


def _edge_prep_body(s_ref, d_ref, os_ref, od_ref):
    i = pl.program_id(0)
    s = _sigma(s_ref[...].reshape(-1))
    d = _sigma(d_ref[...].reshape(-1))
    os_ref[pl.ds(i * _EB, _EB)] = s
    od_ref[pl.ds(i * _EB, _EB)] = d

    @pl.when(i == pl.num_programs(0) - 1)
    def _pad():
        npad = _E_PAD - N_EDGES
        ar = lax.broadcasted_iota(jnp.int32, (npad,), 0)
        os_ref[pl.ds(N_EDGES, npad)] = _sigma(ar % N_NODES)
        od_ref[pl.ds(N_EDGES, npad)] = _sigma(
            N_NODES + ar % (N_PAD - N_NODES))


def _edge_prep(edge_index):
    edge_index3 = edge_index.reshape(2, 1, N_EDGES)
    outs = pl.pallas_call(
        _edge_prep_body,
        grid=(N_EDGES // _EB,),
        in_specs=[
            pl.BlockSpec((1, 1, _EB), lambda i: (0, 0, i)),
            pl.BlockSpec((1, 1, _EB), lambda i: (1, 0, i)),
        ],
        out_specs=[
            pl.BlockSpec((_E_PAD,), lambda i: (0,)),
            pl.BlockSpec((_E_PAD,), lambda i: (0,)),
        ],
        out_shape=[
            jax.ShapeDtypeStruct((_E_PAD,), jnp.int32),
            jax.ShapeDtypeStruct((_E_PAD,), jnp.int32),
        ],
    )(edge_index3, edge_index3)
    return outs


def kernel(x, edge_index, batch, W1a, b1a, W1b, b1b, W2a, b2a, W2b, b2b,
           Wg, bg, Wo, bo):
    src1d, dst1d = _edge_prep(edge_index.astype(jnp.int32))
    src = src1d.reshape(_E_PAD // _CH, _CH)
    dst = dst1d.reshape(_E_PAD // _CH, _CH)
    nb = _HN // _BM
    batch2d = jnp.concatenate(
        [batch.astype(jnp.int32),
         jnp.zeros((N_PAD - N_NODES,), jnp.int32)]).reshape(2 * nb, _BM)

    y1 = _proj(x, W1a)
    p1a, p1b = _edge_agg(y1.reshape(N_PAD, HID), src, dst)
    y2 = _mid(y1, p1a.reshape(_HN, 2 * HID), p1b.reshape(_HN, 2 * HID),
              _pair2(b1a), _blockdiag(W1b), _pair2(b1b), _blockdiag(W2a))
    p2a, p2b = _edge_agg(y2.reshape(N_PAD, HID), src, dst)
    return _tail_stage(y2, p2a.reshape(_HN, 2 * HID), p2b.reshape(_HN, 2 * HID),
                       _pair2(b2a), _blockdiag(W2b), _pair2(b2b),
                       batch2d, Wg, bg.reshape(1, -1), Wo, bo.reshape(1, -1))


# speedup vs baseline: 3.6836x; 1.0343x over previous
import functools

import jax
import jax.numpy as jnp
from jax import lax
from jax.experimental import pallas as pl
from jax.experimental.pallas import tpu as pltpu
from jax.experimental.pallas import tpu_sc as plsc

N_NODES = 10000
N_PAD = 10240
N_EDGES = 320000
HID = 64
N_GRAPHS = 128

_NC, _NS = 2, 16
_NW = _NC * _NS
_CH = 128
_CPT = 80
_E_PAD = _NW * _CPT * _CH
_NBUF = 8
_LA = 4
_RPT = N_PAD // _NS


def _edge_agg(y, src2d, dst2d):
    mesh = plsc.VectorSubcoreMesh(core_axis_name="c", subcore_axis_name="s")

    @functools.partial(
        pl.kernel,
        mesh=mesh,
        compiler_params=pltpu.CompilerParams(use_tc_tiling_on_sc=False),
        out_type=[jax.ShapeDtypeStruct((N_PAD, HID), jnp.float32),
                  jax.ShapeDtypeStruct((N_PAD, HID), jnp.float32)],
        scratch_types=[
            pltpu.VMEM((_CPT, _CH), jnp.int32),
            pltpu.VMEM((_CPT, _CH), jnp.int32),
            [pltpu.VMEM((_CH, HID), jnp.float32) for _ in range(_NBUF)],
            pltpu.VMEM_SHARED((N_PAD, HID), jnp.float32),
            [pltpu.SemaphoreType.DMA for _ in range(_NBUF)],
            [pltpu.SemaphoreType.DMA for _ in range(_NBUF)],
        ],
    )
    def agg(y_hbm, src_hbm, dst_hbm, out0_hbm, out1_hbm,
            sidx, didx, rows, acc, sem_g, sem_s):
        c = lax.axis_index("c")
        s = lax.axis_index("s")
        tid = c * _NS + s

        zeros16 = jnp.zeros((16,), jnp.float32)
        zbuf = rows[0]

        def zrow(r, _):
            for j in range(HID // 16):
                zbuf[r, pl.ds(j * 16, 16)] = zeros16
            return ()

        lax.fori_loop(0, _CH, zrow, ())
        for i in range(_RPT // _CH):
            pltpu.sync_copy(zbuf, acc.at[pl.ds(s * _RPT + i * _CH, _CH)])

        r0 = tid * _CPT
        pltpu.sync_copy(src_hbm.at[pl.ds(r0, _CPT)], sidx)
        pltpu.sync_copy(dst_hbm.at[pl.ds(r0, _CPT)], didx)
        plsc.subcore_barrier()

        def g_start(j, b):
            pltpu.async_copy(y_hbm.at[sidx.at[j]], rows[b], sem_g[b])

        def g_wait(b):
            pltpu.make_async_copy(y_hbm.at[pl.ds(0, _CH)], rows[b],
                                  sem_g[b]).wait()

        def s_start(k, b):
            pltpu.async_copy(rows[b], acc.at[didx.at[k]], sem_s[b], add=True)

        def s_wait(b):
            pltpu.make_async_copy(rows[b], acc.at[pl.ds(0, _CH)],
                                  sem_s[b]).wait()

        for j in range(_LA):
            g_start(j, j)
        for k in range(_LA):
            g_start(k + _LA, k + _LA)
            g_wait(k)
            s_start(k, k)

        def group(g, _):
            for b8 in range(_NBUF):
                k = _LA + g * _NBUF + b8
                bj = b8
                s_wait(bj)
                g_start(k + _LA, bj)
                b = (_LA + b8) % _NBUF
                g_wait(b)
                s_start(k, b)
            return ()

        lax.fori_loop(0, (_CPT - 2 * _LA) // _NBUF, group, ())

        for k in range(_CPT - _LA, _CPT):
            b = k % _NBUF
            g_wait(b)
            s_start(k, b)
        for b in range(_NBUF):
            s_wait(b)

        plsc.subcore_barrier()

        @pl.when(c == 0)
        def _flush0():
            pltpu.sync_copy(acc.at[pl.ds(s * _RPT, _RPT)],
                            out0_hbm.at[pl.ds(s * _RPT, _RPT)])

        @pl.when(c == 1)
        def _flush1():
            pltpu.sync_copy(acc.at[pl.ds(s * _RPT, _RPT)],
                            out1_hbm.at[pl.ds(s * _RPT, _RPT)])

    return agg(y, src2d, dst2d)


_BM = 1024
_HN = N_PAD // 2


def _proj_body(xa_ref, xb_ref, w_ref, o_ref):
    i = pl.program_id(0)
    a = jnp.dot(xa_ref[...], w_ref[...], preferred_element_type=jnp.float32)
    b = jnp.dot(xb_ref[...], w_ref[...], preferred_element_type=jnp.float32)
    nvalid_b = N_NODES - _HN - i * _BM
    valid_b = lax.broadcasted_iota(jnp.int32, (_BM, HID), 0) < nvalid_b
    o_ref[...] = jnp.concatenate([a, jnp.where(valid_b, b, 0.0)], axis=1)


def _proj(x, w):
    k = x.shape[1]
    n = w.shape[1]
    nb = _HN // _BM
    return pl.pallas_call(
        _proj_body,
        grid=(nb,),
        in_specs=[
            pl.BlockSpec((_BM, k), lambda i: (i, 0)),
            pl.BlockSpec((_BM, k), lambda i: (i + _HN // _BM, 0)),
            pl.BlockSpec((k, n), lambda i: (0, 0)),
        ],
        out_specs=pl.BlockSpec((_BM, 2 * n), lambda i: (i, 0)),
        out_shape=jax.ShapeDtypeStruct((_HN, 2 * n), jnp.float32),
    )(x, x, w)


def _mid_body(y1_ref, p0_ref, p1_ref, b1a_ref, w1b_ref, b1b_ref, w2a_ref,
              o_ref):
    u = jnp.maximum(y1_ref[...] + p0_ref[...] + p1_ref[...] + b1a_ref[...],
                    0.0)
    h = jnp.maximum(
        jnp.dot(u, w1b_ref[...], preferred_element_type=jnp.float32)
        + b1b_ref[...], 0.0)
    o_ref[...] = jnp.dot(h, w2a_ref[...], preferred_element_type=jnp.float32)


def _mid(y1, p0, p1, b1a2, w1b_bd, b1b2, w2a_bd):
    return pl.pallas_call(
        _mid_body,
        grid=(_HN // _BM,),
        in_specs=[
            pl.BlockSpec((_BM, 2 * HID), lambda i: (i, 0)),
            pl.BlockSpec((_BM, 2 * HID), lambda i: (i, 0)),
            pl.BlockSpec((_BM, 2 * HID), lambda i: (i, 0)),
            pl.BlockSpec((1, 2 * HID), lambda i: (0, 0)),
            pl.BlockSpec((2 * HID, 2 * HID), lambda i: (0, 0)),
            pl.BlockSpec((1, 2 * HID), lambda i: (0, 0)),
            pl.BlockSpec((2 * HID, 2 * HID), lambda i: (0, 0)),
        ],
        out_specs=pl.BlockSpec((_BM, 2 * HID), lambda i: (i, 0)),
        out_shape=jax.ShapeDtypeStruct((_HN, 2 * HID), jnp.float32),
    )(y1, p0, p1, b1a2, w1b_bd, b1b2, w2a_bd)


def _tail_body(y2_ref, p0_ref, p1_ref, b2a_ref, w2b_ref, b2b_ref, batch_ref,
               wg_ref, bg_ref, wo_ref, bo_ref, g_ref, o_ref):
    i = pl.program_id(0)
    ng = pl.num_programs(0)
    v = jnp.maximum(y2_ref[...] + p0_ref[...] + p1_ref[...] + b2a_ref[...],
                    0.0)
    h2 = jnp.maximum(
        jnp.dot(v, w2b_ref[...], preferred_element_type=jnp.float32)
        + b2b_ref[...], 0.0)
    gio = lax.broadcasted_iota(jnp.int32, (N_GRAPHS, _BM), 0)
    ids_a = batch_ref[pl.ds(i, 1), :]
    ids_b = batch_ref[pl.ds(i + ng, 1), :]
    onehot_a = (jnp.broadcast_to(ids_a, (N_GRAPHS, _BM)) == gio
                ).astype(jnp.float32)
    onehot_b = (jnp.broadcast_to(ids_b, (N_GRAPHS, _BM)) == gio
                ).astype(jnp.float32)
    nvalid_b = N_NODES - _HN - i * _BM
    valid_b = lax.broadcasted_iota(jnp.int32, (_BM, HID), 0) < nvalid_b
    h2b = jnp.where(valid_b, h2[:, HID:], 0.0)
    gpart = (lax.dot_general(onehot_a, h2[:, :HID], (((1,), (0,)), ((), ())),
                             preferred_element_type=jnp.float32)
             + lax.dot_general(onehot_b, h2b, (((1,), (0,)), ((), ())),
                               preferred_element_type=jnp.float32))

    @pl.when(i == 0)
    def _init():
        g_ref[...] = gpart

    @pl.when(i > 0)
    def _accum():
        g_ref[...] += gpart

    @pl.when(i == ng - 1)
    def _head():
        g = g_ref[...]
        t = jnp.maximum(
            jnp.dot(g, wg_ref[...], preferred_element_type=jnp.float32)
            + bg_ref[...], 0.0)
        o_ref[...] = (jnp.dot(t, wo_ref[...],
                              preferred_element_type=jnp.float32)
                      + bo_ref[...])


def _tail_stage(y2, p0, p1, b2a2, w2b_bd, b2b2, batch2d, wg2, bg, wo, bo):
    out_dim = wo.shape[1]
    nb = _HN // _BM
    _, out = pl.pallas_call(
        _tail_body,
        grid=(nb,),
        in_specs=[
            pl.BlockSpec((_BM, 2 * HID), lambda i: (i, 0)),
            pl.BlockSpec((_BM, 2 * HID), lambda i: (i, 0)),
            pl.BlockSpec((_BM, 2 * HID), lambda i: (i, 0)),
            pl.BlockSpec((1, 2 * HID), lambda i: (0, 0)),
            pl.BlockSpec((2 * HID, 2 * HID), lambda i: (0, 0)),
            pl.BlockSpec((1, 2 * HID), lambda i: (0, 0)),
            pl.BlockSpec((2 * nb, _BM), lambda i: (0, 0)),
            pl.BlockSpec((HID, HID), lambda i: (0, 0)),
            pl.BlockSpec((1, HID), lambda i: (0, 0)),
            pl.BlockSpec((HID, out_dim), lambda i: (0, 0)),
            pl.BlockSpec((1, out_dim), lambda i: (0, 0)),
        ],
        out_specs=[
            pl.BlockSpec((N_GRAPHS, HID), lambda i: (0, 0)),
            pl.BlockSpec((N_GRAPHS, out_dim), lambda i: (0, 0)),
        ],
        out_shape=[
            jax.ShapeDtypeStruct((N_GRAPHS, HID), jnp.float32),
            jax.ShapeDtypeStruct((N_GRAPHS, out_dim), jnp.float32),
        ],
    )(y2, p0, p1, b2a2, w2b_bd, b2b2, batch2d, wg2, bg, wo, bo)
    return out


def _pair2(v):
    return jnp.concatenate([v, v]).reshape(1, -1)


def _blockdiag(w):
    z = jnp.zeros_like(w)
    return jnp.concatenate(
        [jnp.concatenate([w, z], axis=1), jnp.concatenate([z, w], axis=1)],
        axis=0)


def _sigma(i):
    return jnp.where(i < _HN, 2 * i, 2 * i - (N_PAD - 1))


_EB = N_EDGES // 5


def _edge_prep_body(e_ref, os_ref, od_ref):
    i = pl.program_id(0)
    s = _sigma(e_ref[0])
    d = _sigma(e_ref[1])
    os_ref[pl.ds(i * _EB, _EB)] = s
    od_ref[pl.ds(i * _EB, _EB)] = d

    @pl.when(i == pl.num_programs(0) - 1)
    def _pad():
        npad = _E_PAD - N_EDGES
        ar = lax.broadcasted_iota(jnp.int32, (npad,), 0)
        os_ref[pl.ds(N_EDGES, npad)] = _sigma(ar % N_NODES)
        od_ref[pl.ds(N_EDGES, npad)] = _sigma(
            N_NODES + ar % (N_PAD - N_NODES))


def _edge_prep(edge_index):
    outs = pl.pallas_call(
        _edge_prep_body,
        grid=(N_EDGES // _EB,),
        in_specs=[
            pl.BlockSpec((2, _EB), lambda i: (0, i)),
        ],
        out_specs=[
            pl.BlockSpec((_E_PAD,), lambda i: (0,)),
            pl.BlockSpec((_E_PAD,), lambda i: (0,)),
        ],
        out_shape=[
            jax.ShapeDtypeStruct((_E_PAD,), jnp.int32),
            jax.ShapeDtypeStruct((_E_PAD,), jnp.int32),
        ],
    )(edge_index)
    return outs


def kernel(x, edge_index, batch, W1a, b1a, W1b, b1b, W2a, b2a, W2b, b2b,
           Wg, bg, Wo, bo):
    src1d, dst1d = _edge_prep(edge_index.astype(jnp.int32))
    src = src1d.reshape(_E_PAD // _CH, _CH)
    dst = dst1d.reshape(_E_PAD // _CH, _CH)
    nb = _HN // _BM
    batch2d = jnp.concatenate(
        [batch.astype(jnp.int32),
         jnp.zeros((N_PAD - N_NODES,), jnp.int32)]).reshape(2 * nb, _BM)

    y1 = _proj(x, W1a)
    p1a, p1b = _edge_agg(y1.reshape(N_PAD, HID), src, dst)
    y2 = _mid(y1, p1a.reshape(_HN, 2 * HID), p1b.reshape(_HN, 2 * HID),
              _pair2(b1a), _blockdiag(W1b), _pair2(b1b), _blockdiag(W2a))
    p2a, p2b = _edge_agg(y2.reshape(N_PAD, HID), src, dst)
    return _tail_stage(y2, p2a.reshape(_HN, 2 * HID), p2b.reshape(_HN, 2 * HID),
                       _pair2(b2a), _blockdiag(W2b), _pair2(b2b),
                       batch2d, Wg, bg.reshape(1, -1), Wo, bo.reshape(1, -1))


# speedup vs baseline: 3.7509x; 1.0183x over previous
import functools

import jax
import jax.numpy as jnp
from jax import lax
from jax.experimental import pallas as pl
from jax.experimental.pallas import tpu as pltpu
from jax.experimental.pallas import tpu_sc as plsc

N_NODES = 10000
N_PAD = 10240
N_EDGES = 320000
HID = 64
N_GRAPHS = 128

_NC, _NS = 2, 16
_NW = _NC * _NS
_CH = 128
_CPT = 80
_E_PAD = _NW * _CPT * _CH
_NBUF = 8
_LA = 4
_RPT = N_PAD // _NS


def _edge_agg(y, src2d, dst2d):
    mesh = plsc.VectorSubcoreMesh(core_axis_name="c", subcore_axis_name="s")

    @functools.partial(
        pl.kernel,
        mesh=mesh,
        compiler_params=pltpu.CompilerParams(use_tc_tiling_on_sc=False),
        out_type=[jax.ShapeDtypeStruct((N_PAD, HID), jnp.float32),
                  jax.ShapeDtypeStruct((N_PAD, HID), jnp.float32)],
        scratch_types=[
            pltpu.VMEM((_CPT, _CH), jnp.int32),
            pltpu.VMEM((_CPT, _CH), jnp.int32),
            [pltpu.VMEM((_CH, HID), jnp.float32) for _ in range(_NBUF)],
            pltpu.VMEM_SHARED((N_PAD, HID), jnp.float32),
            [pltpu.SemaphoreType.DMA for _ in range(_NBUF)],
            [pltpu.SemaphoreType.DMA for _ in range(_NBUF)],
        ],
    )
    def agg(y_hbm, src_hbm, dst_hbm, out0_hbm, out1_hbm,
            sidx, didx, rows, acc, sem_g, sem_s):
        c = lax.axis_index("c")
        s = lax.axis_index("s")
        tid = c * _NS + s

        zeros16 = jnp.zeros((16,), jnp.float32)
        zbuf = rows[0]

        def zrow(r, _):
            for j in range(HID // 16):
                zbuf[r, pl.ds(j * 16, 16)] = zeros16
            return ()

        lax.fori_loop(0, _CH, zrow, ())
        for i in range(_RPT // _CH):
            pltpu.sync_copy(zbuf, acc.at[pl.ds(s * _RPT + i * _CH, _CH)])

        r0 = tid * _CPT
        pltpu.sync_copy(src_hbm.at[pl.ds(r0, _CPT)], sidx)
        pltpu.sync_copy(dst_hbm.at[pl.ds(r0, _CPT)], didx)
        plsc.subcore_barrier()

        def g_start(j, b):
            pltpu.async_copy(y_hbm.at[sidx.at[j]], rows[b], sem_g[b])

        def g_wait(b):
            pltpu.make_async_copy(y_hbm.at[pl.ds(0, _CH)], rows[b],
                                  sem_g[b]).wait()

        def s_start(k, b):
            pltpu.async_copy(rows[b], acc.at[didx.at[k]], sem_s[b], add=True)

        def s_wait(b):
            pltpu.make_async_copy(rows[b], acc.at[pl.ds(0, _CH)],
                                  sem_s[b]).wait()

        for j in range(_LA):
            g_start(j, j)
        for k in range(_LA):
            g_start(k + _LA, k + _LA)
            g_wait(k)
            s_start(k, k)

        def group(g, _):
            for b8 in range(_NBUF):
                k = _LA + g * _NBUF + b8
                bj = b8
                s_wait(bj)
                g_start(k + _LA, bj)
                b = (_LA + b8) % _NBUF
                g_wait(b)
                s_start(k, b)
            return ()

        lax.fori_loop(0, (_CPT - 2 * _LA) // _NBUF, group, ())

        for k in range(_CPT - _LA, _CPT):
            b = k % _NBUF
            g_wait(b)
            s_start(k, b)
        for b in range(_NBUF):
            s_wait(b)

        plsc.subcore_barrier()

        @pl.when(c == 0)
        def _flush0():
            pltpu.sync_copy(acc.at[pl.ds(s * _RPT, _RPT)],
                            out0_hbm.at[pl.ds(s * _RPT, _RPT)])

        @pl.when(c == 1)
        def _flush1():
            pltpu.sync_copy(acc.at[pl.ds(s * _RPT, _RPT)],
                            out1_hbm.at[pl.ds(s * _RPT, _RPT)])

    return agg(y, src2d, dst2d)


_BM = 1024
_HN = N_PAD // 2


_EB = N_EDGES // 5


def _proj_body(xa_ref, xb_ref, w_ref, e_ref, o_ref, os_ref, od_ref):
    i = pl.program_id(0)
    a = jnp.dot(xa_ref[...], w_ref[...], preferred_element_type=jnp.float32)
    b = jnp.dot(xb_ref[...], w_ref[...], preferred_element_type=jnp.float32)
    nvalid_b = N_NODES - _HN - i * _BM
    valid_b = lax.broadcasted_iota(jnp.int32, (_BM, HID), 0) < nvalid_b
    o_ref[...] = jnp.concatenate([a, jnp.where(valid_b, b, 0.0)], axis=1)
    os_ref[pl.ds(i * _EB, _EB)] = _sigma(e_ref[0])
    od_ref[pl.ds(i * _EB, _EB)] = _sigma(e_ref[1])

    @pl.when(i == pl.num_programs(0) - 1)
    def _pad():
        npad = _E_PAD - N_EDGES
        ar = lax.broadcasted_iota(jnp.int32, (npad,), 0)
        os_ref[pl.ds(N_EDGES, npad)] = _sigma(ar % N_NODES)
        od_ref[pl.ds(N_EDGES, npad)] = _sigma(
            N_NODES + ar % (N_PAD - N_NODES))


def _proj(x, w, edge_index):
    k = x.shape[1]
    n = w.shape[1]
    nb = _HN // _BM
    return pl.pallas_call(
        _proj_body,
        grid=(nb,),
        in_specs=[
            pl.BlockSpec((_BM, k), lambda i: (i, 0)),
            pl.BlockSpec((_BM, k), lambda i: (i + _HN // _BM, 0)),
            pl.BlockSpec((k, n), lambda i: (0, 0)),
            pl.BlockSpec((2, _EB), lambda i: (0, i)),
        ],
        out_specs=[
            pl.BlockSpec((_BM, 2 * n), lambda i: (i, 0)),
            pl.BlockSpec((_E_PAD,), lambda i: (0,)),
            pl.BlockSpec((_E_PAD,), lambda i: (0,)),
        ],
        out_shape=[
            jax.ShapeDtypeStruct((_HN, 2 * n), jnp.float32),
            jax.ShapeDtypeStruct((_E_PAD,), jnp.int32),
            jax.ShapeDtypeStruct((_E_PAD,), jnp.int32),
        ],
    )(x, x, w, edge_index)


def _mid_body(y1_ref, p0_ref, p1_ref, b1a_ref, w1b_ref, b1b_ref, w2a_ref,
              o_ref):
    u = jnp.maximum(y1_ref[...] + p0_ref[...] + p1_ref[...] + b1a_ref[...],
                    0.0)
    h = jnp.maximum(
        jnp.dot(u, w1b_ref[...], preferred_element_type=jnp.float32)
        + b1b_ref[...], 0.0)
    o_ref[...] = jnp.dot(h, w2a_ref[...], preferred_element_type=jnp.float32)


def _mid(y1, p0, p1, b1a2, w1b_bd, b1b2, w2a_bd):
    return pl.pallas_call(
        _mid_body,
        grid=(_HN // _BM,),
        in_specs=[
            pl.BlockSpec((_BM, 2 * HID), lambda i: (i, 0)),
            pl.BlockSpec((_BM, 2 * HID), lambda i: (i, 0)),
            pl.BlockSpec((_BM, 2 * HID), lambda i: (i, 0)),
            pl.BlockSpec((1, 2 * HID), lambda i: (0, 0)),
            pl.BlockSpec((2 * HID, 2 * HID), lambda i: (0, 0)),
            pl.BlockSpec((1, 2 * HID), lambda i: (0, 0)),
            pl.BlockSpec((2 * HID, 2 * HID), lambda i: (0, 0)),
        ],
        out_specs=pl.BlockSpec((_BM, 2 * HID), lambda i: (i, 0)),
        out_shape=jax.ShapeDtypeStruct((_HN, 2 * HID), jnp.float32),
    )(y1, p0, p1, b1a2, w1b_bd, b1b2, w2a_bd)


def _tail_body(y2_ref, p0_ref, p1_ref, b2a_ref, w2b_ref, b2b_ref, batch_ref,
               wg_ref, bg_ref, wo_ref, bo_ref, g_ref, o_ref):
    i = pl.program_id(0)
    ng = pl.num_programs(0)
    v = jnp.maximum(y2_ref[...] + p0_ref[...] + p1_ref[...] + b2a_ref[...],
                    0.0)
    h2 = jnp.maximum(
        jnp.dot(v, w2b_ref[...], preferred_element_type=jnp.float32)
        + b2b_ref[...], 0.0)
    gio = lax.broadcasted_iota(jnp.int32, (N_GRAPHS, _BM), 0)
    ids_a = batch_ref[pl.ds(i, 1), :]
    ids_b = batch_ref[pl.ds(i + ng, 1), :]
    onehot_a = (jnp.broadcast_to(ids_a, (N_GRAPHS, _BM)) == gio
                ).astype(jnp.float32)
    onehot_b = (jnp.broadcast_to(ids_b, (N_GRAPHS, _BM)) == gio
                ).astype(jnp.float32)
    nvalid_b = N_NODES - _HN - i * _BM
    valid_b = lax.broadcasted_iota(jnp.int32, (_BM, HID), 0) < nvalid_b
    h2b = jnp.where(valid_b, h2[:, HID:], 0.0)
    gpart = (lax.dot_general(onehot_a, h2[:, :HID], (((1,), (0,)), ((), ())),
                             preferred_element_type=jnp.float32)
             + lax.dot_general(onehot_b, h2b, (((1,), (0,)), ((), ())),
                               preferred_element_type=jnp.float32))

    @pl.when(i == 0)
    def _init():
        g_ref[...] = gpart

    @pl.when(i > 0)
    def _accum():
        g_ref[...] += gpart

    @pl.when(i == ng - 1)
    def _head():
        g = g_ref[...]
        t = jnp.maximum(
            jnp.dot(g, wg_ref[...], preferred_element_type=jnp.float32)
            + bg_ref[...], 0.0)
        o_ref[...] = (jnp.dot(t, wo_ref[...],
                              preferred_element_type=jnp.float32)
                      + bo_ref[...])


def _tail_stage(y2, p0, p1, b2a2, w2b_bd, b2b2, batch2d, wg2, bg, wo, bo):
    out_dim = wo.shape[1]
    nb = _HN // _BM
    _, out = pl.pallas_call(
        _tail_body,
        grid=(nb,),
        in_specs=[
            pl.BlockSpec((_BM, 2 * HID), lambda i: (i, 0)),
            pl.BlockSpec((_BM, 2 * HID), lambda i: (i, 0)),
            pl.BlockSpec((_BM, 2 * HID), lambda i: (i, 0)),
            pl.BlockSpec((1, 2 * HID), lambda i: (0, 0)),
            pl.BlockSpec((2 * HID, 2 * HID), lambda i: (0, 0)),
            pl.BlockSpec((1, 2 * HID), lambda i: (0, 0)),
            pl.BlockSpec((2 * nb, _BM), lambda i: (0, 0)),
            pl.BlockSpec((HID, HID), lambda i: (0, 0)),
            pl.BlockSpec((1, HID), lambda i: (0, 0)),
            pl.BlockSpec((HID, out_dim), lambda i: (0, 0)),
            pl.BlockSpec((1, out_dim), lambda i: (0, 0)),
        ],
        out_specs=[
            pl.BlockSpec((N_GRAPHS, HID), lambda i: (0, 0)),
            pl.BlockSpec((N_GRAPHS, out_dim), lambda i: (0, 0)),
        ],
        out_shape=[
            jax.ShapeDtypeStruct((N_GRAPHS, HID), jnp.float32),
            jax.ShapeDtypeStruct((N_GRAPHS, out_dim), jnp.float32),
        ],
    )(y2, p0, p1, b2a2, w2b_bd, b2b2, batch2d, wg2, bg, wo, bo)
    return out


def _pair2(v):
    return jnp.concatenate([v, v]).reshape(1, -1)


def _blockdiag(w):
    z = jnp.zeros_like(w)
    return jnp.concatenate(
        [jnp.concatenate([w, z], axis=1), jnp.concatenate([z, w], axis=1)],
        axis=0)


def _sigma(i):
    return jnp.where(i < _HN, 2 * i, 2 * i - (N_PAD - 1))


_EB = N_EDGES // 5


def _proj_body(xa_ref, xb_ref, w_ref, e_ref, o_ref, os_ref, od_ref):
    i = pl.program_id(0)
    a = jnp.dot(xa_ref[...], w_ref[...], preferred_element_type=jnp.float32)
    b = jnp.dot(xb_ref[...], w_ref[...], preferred_element_type=jnp.float32)
    nvalid_b = N_NODES - _HN - i * _BM
    valid_b = lax.broadcasted_iota(jnp.int32, (_BM, HID), 0) < nvalid_b
    o_ref[...] = jnp.concatenate([a, jnp.where(valid_b, b, 0.0)], axis=1)
    os_ref[pl.ds(i * _EB, _EB)] = _sigma(e_ref[0])
    od_ref[pl.ds(i * _EB, _EB)] = _sigma(e_ref[1])

    @pl.when(i == pl.num_programs(0) - 1)
    def _pad():
        npad = _E_PAD - N_EDGES
        ar = lax.broadcasted_iota(jnp.int32, (npad,), 0)
        os_ref[pl.ds(N_EDGES, npad)] = _sigma(ar % N_NODES)
        od_ref[pl.ds(N_EDGES, npad)] = _sigma(
            N_NODES + ar % (N_PAD - N_NODES))


def _proj(x, w, edge_index):
    k = x.shape[1]
    n = w.shape[1]
    nb = _HN // _BM
    return pl.pallas_call(
        _proj_body,
        grid=(nb,),
        in_specs=[
            pl.BlockSpec((_BM, k), lambda i: (i, 0)),
            pl.BlockSpec((_BM, k), lambda i: (i + _HN // _BM, 0)),
            pl.BlockSpec((k, n), lambda i: (0, 0)),
            pl.BlockSpec((2, _EB), lambda i: (0, i)),
        ],
        out_specs=[
            pl.BlockSpec((_BM, 2 * n), lambda i: (i, 0)),
            pl.BlockSpec((_E_PAD,), lambda i: (0,)),
            pl.BlockSpec((_E_PAD,), lambda i: (0,)),
        ],
        out_shape=[
            jax.ShapeDtypeStruct((_HN, 2 * n), jnp.float32),
            jax.ShapeDtypeStruct((_E_PAD,), jnp.int32),
            jax.ShapeDtypeStruct((_E_PAD,), jnp.int32),
        ],
    )(x, x, w, edge_index)


def _mid_body(y1_ref, p0_ref, p1_ref, b1a_ref, w1b_ref, b1b_ref, w2a_ref,
              o_ref):
    u = jnp.maximum(y1_ref[...] + p0_ref[...] + p1_ref[...] + b1a_ref[...],
                    0.0)
    h = jnp.maximum(
        jnp.dot(u, w1b_ref[...], preferred_element_type=jnp.float32)
        + b1b_ref[...], 0.0)
    o_ref[...] = jnp.dot(h, w2a_ref[...], preferred_element_type=jnp.float32)


def _mid(y1, p0, p1, b1a2, w1b_bd, b1b2, w2a_bd):
    return pl.pallas_call(
        _mid_body,
        grid=(_HN // _BM,),
        in_specs=[
            pl.BlockSpec((_BM, 2 * HID), lambda i: (i, 0)),
            pl.BlockSpec((_BM, 2 * HID), lambda i: (i, 0)),
            pl.BlockSpec((_BM, 2 * HID), lambda i: (i, 0)),
            pl.BlockSpec((1, 2 * HID), lambda i: (0, 0)),
            pl.BlockSpec((2 * HID, 2 * HID), lambda i: (0, 0)),
            pl.BlockSpec((1, 2 * HID), lambda i: (0, 0)),
            pl.BlockSpec((2 * HID, 2 * HID), lambda i: (0, 0)),
        ],
        out_specs=pl.BlockSpec((_BM, 2 * HID), lambda i: (i, 0)),
        out_shape=jax.ShapeDtypeStruct((_HN, 2 * HID), jnp.float32),
    )(y1, p0, p1, b1a2, w1b_bd, b1b2, w2a_bd)


def _tail_body(y2_ref, p0_ref, p1_ref, b2a_ref, w2b_ref, b2b_ref, batch_ref,
               wg_ref, bg_ref, wo_ref, bo_ref, g_ref, o_ref):
    i = pl.program_id(0)
    ng = pl.num_programs(0)
    v = jnp.maximum(y2_ref[...] + p0_ref[...] + p1_ref[...] + b2a_ref[...],
                    0.0)
    h2 = jnp.maximum(
        jnp.dot(v, w2b_ref[...], preferred_element_type=jnp.float32)
        + b2b_ref[...], 0.0)
    gio = lax.broadcasted_iota(jnp.int32, (N_GRAPHS, _BM), 0)
    ids_a = batch_ref[pl.ds(i, 1), :]
    ids_b = batch_ref[pl.ds(i + ng, 1), :]
    onehot_a = (jnp.broadcast_to(ids_a, (N_GRAPHS, _BM)) == gio
                ).astype(jnp.float32)
    onehot_b = (jnp.broadcast_to(ids_b, (N_GRAPHS, _BM)) == gio
                ).astype(jnp.float32)
    nvalid_b = N_NODES - _HN - i * _BM
    valid_b = lax.broadcasted_iota(jnp.int32, (_BM, HID), 0) < nvalid_b
    h2b = jnp.where(valid_b, h2[:, HID:], 0.0)
    gpart = (lax.dot_general(onehot_a, h2[:, :HID], (((1,), (0,)), ((), ())),
                             preferred_element_type=jnp.float32)
             + lax.dot_general(onehot_b, h2b, (((1,), (0,)), ((), ())),
                               preferred_element_type=jnp.float32))

    @pl.when(i == 0)
    def _init():
        g_ref[...] = gpart

    @pl.when(i > 0)
    def _accum():
        g_ref[...] += gpart

    @pl.when(i == ng - 1)
    def _head():
        g = g_ref[...]
        t = jnp.maximum(
            jnp.dot(g, wg_ref[...], preferred_element_type=jnp.float32)
            + bg_ref[...], 0.0)
        o_ref[...] = (jnp.dot(t, wo_ref[...],
                              preferred_element_type=jnp.float32)
                      + bo_ref[...])


def _tail_stage(y2, p0, p1, b2a2, w2b_bd, b2b2, batch2d, wg2, bg, wo, bo):
    out_dim = wo.shape[1]
    nb = _HN // _BM
    _, out = pl.pallas_call(
        _tail_body,
        grid=(nb,),
        in_specs=[
            pl.BlockSpec((_BM, 2 * HID), lambda i: (i, 0)),
            pl.BlockSpec((_BM, 2 * HID), lambda i: (i, 0)),
            pl.BlockSpec((_BM, 2 * HID), lambda i: (i, 0)),
            pl.BlockSpec((1, 2 * HID), lambda i: (0, 0)),
            pl.BlockSpec((2 * HID, 2 * HID), lambda i: (0, 0)),
            pl.BlockSpec((1, 2 * HID), lambda i: (0, 0)),
            pl.BlockSpec((2 * nb, _BM), lambda i: (0, 0)),
            pl.BlockSpec((HID, HID), lambda i: (0, 0)),
            pl.BlockSpec((1, HID), lambda i: (0, 0)),
            pl.BlockSpec((HID, out_dim), lambda i: (0, 0)),
            pl.BlockSpec((1, out_dim), lambda i: (0, 0)),
        ],
        out_specs=[
            pl.BlockSpec((N_GRAPHS, HID), lambda i: (0, 0)),
            pl.BlockSpec((N_GRAPHS, out_dim), lambda i: (0, 0)),
        ],
        out_shape=[
            jax.ShapeDtypeStruct((N_GRAPHS, HID), jnp.float32),
            jax.ShapeDtypeStruct((N_GRAPHS, out_dim), jnp.float32),
        ],
    )(y2, p0, p1, b2a2, w2b_bd, b2b2, batch2d, wg2, bg, wo, bo)
    return out


def _pair2(v):
    return jnp.concatenate([v, v]).reshape(1, -1)


def _blockdiag(w):
    z = jnp.zeros_like(w)
    return jnp.concatenate(
        [jnp.concatenate([w, z], axis=1), jnp.concatenate([z, w], axis=1)],
        axis=0)


def _sigma(i):
    return jnp.where(i < _HN, 2 * i, 2 * i - (N_PAD - 1))


_EB = N_EDGES // 5


def _edge_prep_body(e_ref, os_ref, od_ref):
    i = pl.program_id(0)
    s = _sigma(e_ref[0])
    d = _sigma(e_ref[1])
    os_ref[pl.ds(i * _EB, _EB)] = s
    od_ref[pl.ds(i * _EB, _EB)] = d

    @pl.when(i == pl.num_programs(0) - 1)
    def _pad():
        npad = _E_PAD - N_EDGES
        ar = lax.broadcasted_iota(jnp.int32, (npad,), 0)
        os_ref[pl.ds(N_EDGES, npad)] = _sigma(ar % N_NODES)
        od_ref[pl.ds(N_EDGES, npad)] = _sigma(
            N_NODES + ar % (N_PAD - N_NODES))


def _edge_prep(edge_index):
    outs = pl.pallas_call(
        _edge_prep_body,
        grid=(N_EDGES // _EB,),
        in_specs=[
            pl.BlockSpec((2, _EB), lambda i: (0, i)),
        ],
        out_specs=[
            pl.BlockSpec((_E_PAD,), lambda i: (0,)),
            pl.BlockSpec((_E_PAD,), lambda i: (0,)),
        ],
        out_shape=[
            jax.ShapeDtypeStruct((_E_PAD,), jnp.int32),
            jax.ShapeDtypeStruct((_E_PAD,), jnp.int32),
        ],
    )(edge_index)
    return outs


def kernel(x, edge_index, batch, W1a, b1a, W1b, b1b, W2a, b2a, W2b, b2b,
           Wg, bg, Wo, bo):
    ei = edge_index.astype(jnp.int32)
    nb = _HN // _BM
    batch2d = jnp.concatenate(
        [batch.astype(jnp.int32),
         jnp.zeros((N_PAD - N_NODES,), jnp.int32)]).reshape(2 * nb, _BM)

    y1, src1d, dst1d = _proj(x, W1a, ei)
    src = src1d.reshape(_E_PAD // _CH, _CH)
    dst = dst1d.reshape(_E_PAD // _CH, _CH)
    p1a, p1b = _edge_agg(y1.reshape(N_PAD, HID), src, dst)
    y2 = _mid(y1, p1a.reshape(_HN, 2 * HID), p1b.reshape(_HN, 2 * HID),
              _pair2(b1a), _blockdiag(W1b), _pair2(b1b), _blockdiag(W2a))
    p2a, p2b = _edge_agg(y2.reshape(N_PAD, HID), src, dst)
    return _tail_stage(y2, p2a.reshape(_HN, 2 * HID), p2b.reshape(_HN, 2 * HID),
                       _pair2(b2a), _blockdiag(W2b), _pair2(b2b),
                       batch2d, Wg, bg.reshape(1, -1), Wo, bo.reshape(1, -1))
